# baseline (speedup 1.0000x reference)
import functools

import jax
import jax.numpy as jnp
from jax import lax
from jax.experimental import pallas as pl
from jax.experimental.pallas import tpu as pltpu
from jax.experimental.pallas import tpu_sc as plsc

_E = 64
_K = 8
_COEF = 0.01
_NC = 2
_NS = 16
_NW = _NC * _NS
_LANES = 16
_CH = 256
_RPAD = 17
_TILE = _E * _RPAD

_SORT8 = [(0, 1), (2, 3), (4, 5), (6, 7),
          (0, 2), (1, 3), (4, 6), (5, 7),
          (1, 2), (5, 6),
          (0, 4), (1, 5), (2, 6), (3, 7),
          (2, 4), (3, 5),
          (1, 2), (3, 4), (5, 6)]
_BITONIC8 = [(0, 4), (1, 5), (2, 6), (3, 7),
             (0, 2), (1, 3), (4, 6), (5, 7),
             (0, 1), (2, 3), (4, 5), (6, 7)]


def _ce(lst, i, j):
    hi = jnp.maximum(lst[i], lst[j])
    lo = jnp.minimum(lst[i], lst[j])
    lst[i], lst[j] = hi, lo


def _merge_top8(a, b):
    m = [jnp.maximum(a[i], b[7 - i]) for i in range(8)]
    for i, j in _BITONIC8:
        _ce(m, i, j)
    return m


def _sc_loss_partials(gl, base_row, sc_rows, num_cores):
    slab_rows = gl.shape[1]
    n_workers = num_cores * _NS
    rows_per_w = sc_rows // n_workers
    n_chunks = rows_per_w // _CH

    mesh = plsc.VectorSubcoreMesh(
        core_axis_name="c", subcore_axis_name="s",
        num_cores=num_cores, num_subcores=_NS)

    @functools.partial(
        pl.kernel,
        out_type=jax.ShapeDtypeStruct((n_workers, _LANES), jnp.float32),
        mesh=mesh,
        compiler_params=pltpu.CompilerParams(needs_layout_passes=False),
        scratch_types=[
            pltpu.VMEM((_CH, _E), jnp.float32),
            pltpu.VMEM((_CH, _E), jnp.float32),
            pltpu.VMEM((2 * _TILE,), jnp.float32),
            pltpu.VMEM((_LANES,), jnp.float32),
            pltpu.SemaphoreType.DMA,
            pltpu.SemaphoreType.DMA,
        ],
    )
    def body(gl_hbm, out_hbm, buf_a, buf_b, buft, accv, sem_a, sem_b):
        cid = lax.axis_index("c")
        sid = lax.axis_index("s")
        wid = sid * num_cores + cid
        grow = base_row + wid * rows_per_w
        slab = grow // slab_rows
        row0 = grow % slab_rows

        iota = lax.iota(jnp.int32, _LANES)
        s17 = iota * _RPAD

        bufs = [buf_a, buf_b]
        sems = [sem_a, sem_b]

        def start_chunk(ci, slot):
            return pltpu.async_copy(
                gl_hbm.at[slab, pl.ds(row0 + ci * _CH, _CH), :],
                bufs[slot], sems[slot])

        def repack(buf, grow0, toff):
            for r in range(_LANES):
                for gg in range(_E // _LANES):
                    v = buf[grow0 + r, pl.ds(gg * _LANES, _LANES)]
                    plsc.store_scatter(
                        buft, [s17 + (toff + gg * _LANES * _RPAD + r)], v)

        def top8_rp(toff):
            run = None
            for g8 in range(_E // _K):
                sub = [buft[pl.ds(toff + (g8 * _K + t) * _RPAD, _LANES)]
                       for t in range(_K)]
                for i, j in _SORT8:
                    _ce(sub, i, j)
                run = sub if run is None else _merge_top8(run, sub)
            m = run
            s = [jnp.float32(1.0) + jnp.zeros((_LANES,), jnp.float32)] + [
                jnp.exp(mj - m[0]) for mj in m[1:]]
            tot = s[0]
            for j in range(1, _K):
                tot = tot + s[j]
            p = s[0] / tot
            for j in range(1, _K):
                p = p + s[j] / tot
            return p * (1.0 / _K)

        def group_body_for(buf):
            def group_body(gi, acc2):
                repack(buf, gi * 2 * _LANES, 0)
                repack(buf, gi * 2 * _LANES + _LANES, _TILE)
                rp0 = top8_rp(0)
                rp1 = top8_rp(_TILE)
                return acc2 + (rp0 + rp1)

            return group_body

        acc = jnp.zeros((_LANES,), jnp.float32)
        iters_per_chunk = _CH // (2 * _LANES)
        copies = [None, None]
        copies[0] = start_chunk(0, 0)
        for ci in range(n_chunks):
            slot = ci % 2
            if ci + 1 < n_chunks:
                copies[1 - slot] = start_chunk(ci + 1, 1 - slot)
            copies[slot].wait()
            acc = lax.fori_loop(0, iters_per_chunk,
                                group_body_for(bufs[slot]), acc)
        accv[...] = acc
        pltpu.sync_copy(accv, out_hbm.at[wid])

    return body(gl)


def kernel(gate_logits):
    n_rows = gate_logits.size // _E
    half = n_rows // 2
    parts0 = _sc_loss_partials(gate_logits, 0, half, 1)
    parts1 = _sc_loss_partials(gate_logits, half, half, 1)
    scale = _COEF * (_E * _E) / (n_rows * _E)
    return (jnp.sum(parts0) + jnp.sum(parts1)) * jnp.float32(scale)


# speedup vs baseline: 1.7528x; 1.7528x over previous
import functools

import jax
import jax.numpy as jnp
from jax import lax
from jax.experimental import pallas as pl
from jax.experimental.pallas import tpu as pltpu
from jax.experimental.pallas import tpu_sc as plsc

_E = 64
_K = 8
_COEF = 0.01
_NC = 2
_NS = 16
_NW = _NC * _NS
_LANES = 16
_CH = 256
_RPAD = 17
_TILE = _E * _RPAD

_SORT8 = [(0, 1), (2, 3), (4, 5), (6, 7),
          (0, 2), (1, 3), (4, 6), (5, 7),
          (1, 2), (5, 6),
          (0, 4), (1, 5), (2, 6), (3, 7),
          (2, 4), (3, 5),
          (1, 2), (3, 4), (5, 6)]
_BITONIC8 = [(0, 4), (1, 5), (2, 6), (3, 7),
             (0, 2), (1, 3), (4, 6), (5, 7),
             (0, 1), (2, 3), (4, 5), (6, 7)]


def _ce(lst, i, j):
    hi = jnp.maximum(lst[i], lst[j])
    lo = jnp.minimum(lst[i], lst[j])
    lst[i], lst[j] = hi, lo


def _merge_top8(a, b):
    m = [jnp.maximum(a[i], b[7 - i]) for i in range(8)]
    for i, j in _BITONIC8:
        _ce(m, i, j)
    return m


def _sc_loss_partials(gl, base_row, sc_rows, num_cores):
    slab_rows = gl.shape[1]
    n_workers = num_cores * _NS
    rows_per_w = sc_rows // n_workers
    n_chunks = rows_per_w // _CH

    mesh = plsc.VectorSubcoreMesh(
        core_axis_name="c", subcore_axis_name="s",
        num_cores=num_cores, num_subcores=_NS)

    @functools.partial(
        pl.kernel,
        out_type=jax.ShapeDtypeStruct((n_workers, _LANES), jnp.float32),
        mesh=mesh,
        compiler_params=pltpu.CompilerParams(needs_layout_passes=False),
        scratch_types=[
            pltpu.VMEM((_CH, _E), jnp.float32),
            pltpu.VMEM((_CH, _E), jnp.float32),
            pltpu.VMEM((2 * _TILE,), jnp.float32),
            pltpu.VMEM((_LANES,), jnp.float32),
            pltpu.SemaphoreType.DMA,
            pltpu.SemaphoreType.DMA,
        ],
    )
    def body(gl_hbm, out_hbm, buf_a, buf_b, buft, accv, sem_a, sem_b):
        cid = lax.axis_index("c")
        sid = lax.axis_index("s")
        wid = sid * num_cores + cid
        grow = base_row + wid * rows_per_w
        slab = grow // slab_rows
        row0 = grow % slab_rows

        iota = lax.iota(jnp.int32, _LANES)
        s17 = iota * _RPAD

        bufs = [buf_a, buf_b]
        sems = [sem_a, sem_b]

        def start_chunk(ci, slot):
            return pltpu.async_copy(
                gl_hbm.at[slab, pl.ds(row0 + ci * _CH, _CH), :],
                bufs[slot], sems[slot])

        def repack(buf, grow0, toff):
            for r in range(_LANES):
                for gg in range(_E // _LANES):
                    v = buf[grow0 + r, pl.ds(gg * _LANES, _LANES)]
                    plsc.store_scatter(
                        buft, [s17 + (toff + gg * _LANES * _RPAD + r)], v)

        def top8_rp(toff):
            run = None
            for g8 in range(_E // _K):
                sub = [buft[pl.ds(toff + (g8 * _K + t) * _RPAD, _LANES)]
                       for t in range(_K)]
                for i, j in _SORT8:
                    _ce(sub, i, j)
                run = sub if run is None else _merge_top8(run, sub)
            m = run
            s = [jnp.float32(1.0) + jnp.zeros((_LANES,), jnp.float32)] + [
                jnp.exp(mj - m[0]) for mj in m[1:]]
            tot = s[0]
            for j in range(1, _K):
                tot = tot + s[j]
            p = s[0] / tot
            for j in range(1, _K):
                p = p + s[j] / tot
            return p * (1.0 / _K)

        def group_body_for(buf):
            def group_body(gi, acc2):
                repack(buf, gi * 2 * _LANES, 0)
                repack(buf, gi * 2 * _LANES + _LANES, _TILE)
                rp0 = top8_rp(0)
                rp1 = top8_rp(_TILE)
                return acc2 + (rp0 + rp1)

            return group_body

        acc = jnp.zeros((_LANES,), jnp.float32)
        iters_per_chunk = _CH // (2 * _LANES)
        copies = [None, None]
        copies[0] = start_chunk(0, 0)
        for ci in range(n_chunks):
            slot = ci % 2
            if ci + 1 < n_chunks:
                copies[1 - slot] = start_chunk(ci + 1, 1 - slot)
            copies[slot].wait()
            acc = lax.fori_loop(0, iters_per_chunk,
                                group_body_for(bufs[slot]), acc)
        accv[...] = acc
        pltpu.sync_copy(accv, out_hbm.at[wid])

    return body(gl)


_TCB = 512


def _tc_loss_partial(gl2, tc_rows):
    nblk = tc_rows // _TCB

    def tc_body(x_ref, out_ref):
        x = x_ref[...]
        w = x.T
        maxes = []
        for k in range(_K):
            mk = jnp.max(w, axis=0)
            if k + 1 < _K:
                w = jnp.where(w == mk[None, :], -jnp.inf, w)
            maxes.append(mk)
        s = [jnp.exp(mj - maxes[0]) for mj in maxes]
        tot = s[0]
        for j in range(1, _K):
            tot = tot + s[j]
        p = s[0] / tot
        for j in range(1, _K):
            p = p + s[j] / tot
        part = (jnp.sum(p) * (1.0 / _K)).reshape(1, 1)

        @pl.when(pl.program_id(0) == 0)
        def _():
            out_ref[...] = jnp.zeros((1, 1), jnp.float32)

        out_ref[...] += part

    return pl.pallas_call(
        tc_body,
        grid=(nblk,),
        in_specs=[pl.BlockSpec((_TCB, _E), lambda i: (i, 0))],
        out_specs=pl.BlockSpec((1, 1), lambda i: (0, 0)),
        out_shape=jax.ShapeDtypeStruct((1, 1), jnp.float32),
    )(gl2)


_TC_FRac_SLABS = 2


def kernel(gate_logits):
    n_rows = gate_logits.size // _E
    slab_rows = gate_logits.shape[1]
    tc_rows = _TC_FRac_SLABS * slab_rows
    sc_rows = n_rows - tc_rows
    parts = _sc_loss_partials(gate_logits, tc_rows, sc_rows, _NC)
    tc_part = _tc_loss_partial(gate_logits.reshape(-1, _E), tc_rows)
    scale = _COEF * (_E * _E) / (n_rows * _E)
    return (jnp.sum(parts) + tc_part[0, 0]) * jnp.float32(scale)


# speedup vs baseline: 1.7560x; 1.0019x over previous
import functools

import jax
import jax.numpy as jnp
from jax import lax
from jax.experimental import pallas as pl
from jax.experimental.pallas import tpu as pltpu
from jax.experimental.pallas import tpu_sc as plsc

_E = 64
_K = 8
_COEF = 0.01
_NC = 2
_NS = 16
_NW = _NC * _NS
_LANES = 16
_CH = 256
_RPAD = 17
_TILE = _E * _RPAD

_SORT8 = [(0, 1), (2, 3), (4, 5), (6, 7),
          (0, 2), (1, 3), (4, 6), (5, 7),
          (1, 2), (5, 6),
          (0, 4), (1, 5), (2, 6), (3, 7),
          (2, 4), (3, 5),
          (1, 2), (3, 4), (5, 6)]
_BITONIC8 = [(0, 4), (1, 5), (2, 6), (3, 7),
             (0, 2), (1, 3), (4, 6), (5, 7),
             (0, 1), (2, 3), (4, 5), (6, 7)]


def _ce(lst, i, j):
    hi = jnp.maximum(lst[i], lst[j])
    lo = jnp.minimum(lst[i], lst[j])
    lst[i], lst[j] = hi, lo


def _merge_top8(a, b):
    m = [jnp.maximum(a[i], b[7 - i]) for i in range(8)]
    for i, j in _BITONIC8:
        _ce(m, i, j)
    return m


def _sc_loss_partials(gl, base_row, sc_rows, num_cores):
    slab_rows = gl.shape[1]
    n_workers = num_cores * _NS
    rows_per_w = sc_rows // n_workers
    n_chunks = rows_per_w // _CH

    mesh = plsc.VectorSubcoreMesh(
        core_axis_name="c", subcore_axis_name="s",
        num_cores=num_cores, num_subcores=_NS)

    @functools.partial(
        pl.kernel,
        out_type=jax.ShapeDtypeStruct((n_workers, _LANES), jnp.float32),
        mesh=mesh,
        compiler_params=pltpu.CompilerParams(needs_layout_passes=False),
        scratch_types=[
            pltpu.VMEM((_CH, _E), jnp.float32),
            pltpu.VMEM((_CH, _E), jnp.float32),
            pltpu.VMEM((2 * _TILE,), jnp.float32),
            pltpu.VMEM((_LANES,), jnp.float32),
            pltpu.SemaphoreType.DMA,
            pltpu.SemaphoreType.DMA,
        ],
    )
    def body(gl_hbm, out_hbm, buf_a, buf_b, buft, accv, sem_a, sem_b):
        cid = lax.axis_index("c")
        sid = lax.axis_index("s")
        wid = sid * num_cores + cid
        grow = base_row + wid * rows_per_w
        slab = grow // slab_rows
        row0 = grow % slab_rows

        iota = lax.iota(jnp.int32, _LANES)
        s17 = iota * _RPAD

        bufs = [buf_a, buf_b]
        sems = [sem_a, sem_b]

        def start_chunk(ci, slot):
            return pltpu.async_copy(
                gl_hbm.at[slab, pl.ds(row0 + ci * _CH, _CH), :],
                bufs[slot], sems[slot])

        def repack(buf, grow0, toff):
            for r in range(_LANES):
                for gg in range(_E // _LANES):
                    v = buf[grow0 + r, pl.ds(gg * _LANES, _LANES)]
                    plsc.store_scatter(
                        buft, [s17 + (toff + gg * _LANES * _RPAD + r)], v)

        def top8_rp(toff):
            run = None
            for g8 in range(_E // _K):
                sub = [buft[pl.ds(toff + (g8 * _K + t) * _RPAD, _LANES)]
                       for t in range(_K)]
                for i, j in _SORT8:
                    _ce(sub, i, j)
                run = sub if run is None else _merge_top8(run, sub)
            m = run
            s = [jnp.float32(1.0) + jnp.zeros((_LANES,), jnp.float32)] + [
                jnp.exp(mj - m[0]) for mj in m[1:]]
            tot = s[0]
            for j in range(1, _K):
                tot = tot + s[j]
            p = s[0] / tot
            for j in range(1, _K):
                p = p + s[j] / tot
            return p * (1.0 / _K)

        def group_body_for(buf):
            def group_body(gi, acc2):
                repack(buf, gi * 2 * _LANES, 0)
                repack(buf, gi * 2 * _LANES + _LANES, _TILE)
                rp0 = top8_rp(0)
                rp1 = top8_rp(_TILE)
                return acc2 + (rp0 + rp1)

            return group_body

        acc = jnp.zeros((_LANES,), jnp.float32)
        iters_per_chunk = _CH // (2 * _LANES)
        copies = [None, None]
        copies[0] = start_chunk(0, 0)
        for ci in range(n_chunks):
            slot = ci % 2
            if ci + 1 < n_chunks:
                copies[1 - slot] = start_chunk(ci + 1, 1 - slot)
            copies[slot].wait()
            acc = lax.fori_loop(0, iters_per_chunk,
                                group_body_for(bufs[slot]), acc)
        accv[...] = acc
        pltpu.sync_copy(accv, out_hbm.at[wid])

    return body(gl)


_TCB = 512


def _tc_loss_partial(gl, tc_rows):
    bps = gl.shape[1] // _TCB
    nblk = tc_rows // _TCB

    def _rp_sum(w):
        maxes = []
        for k in range(_K):
            mk = jnp.max(w, axis=0)
            if k + 1 < _K:
                w = jnp.where(w == mk[None, :], -jnp.inf, w)
            maxes.append(mk)
        s = [jnp.exp(mj - maxes[0]) for mj in maxes]
        tot = s[0]
        for j in range(1, _K):
            tot = tot + s[j]
        p = s[0] / tot
        for j in range(1, _K):
            p = p + s[j] / tot
        return jnp.sum(p) * (1.0 / _K)

    def tc_body(x_ref, out_ref):
        x = x_ref[0]
        ra = _rp_sum(x[: _TCB // 2].T)
        rb = _rp_sum(x[_TCB // 2:].T)
        part = (ra + rb).reshape(1, 1)

        @pl.when(pl.program_id(0) == 0)
        def _():
            out_ref[...] = jnp.zeros((1, 1), jnp.float32)

        out_ref[...] += part

    return pl.pallas_call(
        tc_body,
        grid=(nblk,),
        in_specs=[pl.BlockSpec((1, _TCB, _E),
                               lambda i: (i // bps, i % bps, 0))],
        out_specs=pl.BlockSpec((1, 1), lambda i: (0, 0)),
        out_shape=jax.ShapeDtypeStruct((1, 1), jnp.float32),
    )(gl)


_TC_FRac_SLABS = 2


def kernel(gate_logits):
    n_rows = gate_logits.size // _E
    slab_rows = gate_logits.shape[1]
    tc_rows = _TC_FRac_SLABS * slab_rows
    sc_rows = n_rows - tc_rows
    parts = _sc_loss_partials(gate_logits, tc_rows, sc_rows, _NC)
    tc_part = _tc_loss_partial(gate_logits, tc_rows)
    scale = _COEF * (_E * _E) / (n_rows * _E)
    return (jnp.sum(parts) + tc_part[0, 0]) * jnp.float32(scale)
